# baseline (speedup 1.0000x reference)
import functools

import jax
import jax.numpy as jnp
from jax import lax
from jax.experimental import pallas as pl
from jax.experimental.pallas import tpu as pltpu
from jax.experimental.pallas import tpu_sc as plsc

_PERCENT_T = 0.02
_N = 512 * 512
_K = int(round(_N * _PERCENT_T))
_NROWS = 64
_NTILES = 32
_ROWS_PER_TILE = _NROWS // _NTILES
_CH = 32768
_NCHUNK = _N // _CH
_L12_BINS = 2048
_L3_BINS = 1024
_HIST_WORDS = _L12_BINS * 16


def _monotone_key(x):
    b = lax.bitcast_convert_type(x, jnp.int32)
    return jnp.where(b < 0, b ^ jnp.int32(0x7FFFFFFF), b)


def _decode_key(key):
    fb = jnp.where(key < 0, key ^ jnp.int32(0x7FFFFFFF), key)
    return lax.bitcast_convert_type(fb, jnp.float32)


def _sc_body(cam_ref, out_ref, hist, chunk, outv):
    cid = lax.axis_index("c")
    sid = lax.axis_index("s")
    wid = sid * 2 + cid
    lane = lax.iota(jnp.int32, 16)
    ones = jnp.ones((16,), jnp.int32)
    zeros_i = jnp.zeros((16,), jnp.int32)
    zeros_f = jnp.zeros((16,), jnp.float32)

    def zero_hist(nwords):
        def zb(i, carry):
            hist[pl.ds(i * 16, 16)] = zeros_i
            return carry
        lax.fori_loop(0, nwords // 16, zb, 0)

    def scan_down(start_bin, target):
        def cond(c):
            _, cum, _ = c
            return cum < target

        def body(c):
            b, cum, _ = c
            cnt = jnp.sum(hist[pl.ds(b * 16, 16)])
            return (b - 1, cum + cnt, cnt)

        b, cum, last = lax.while_loop(
            cond, body, (jnp.int32(start_bin), jnp.int32(0), jnp.int32(0)))
        return b + 1, cum - last

    def run_row(row):
        zero_hist(_HIST_WORDS)
        for ci in range(_NCHUNK):
            pltpu.sync_copy(cam_ref.at[row, ci], chunk)

            def b1(i, carry):
                x = chunk[pl.ds(i * 16, 16)]
                key = _monotone_key(x)
                bin1 = lax.shift_right_arithmetic(key, 21) + jnp.int32(1024)
                plsc.addupdate_scatter(hist, [bin1 * 16 + lane], ones)
                return carry
            lax.fori_loop(0, _CH // 16, b1, 0)
        h1, above1 = scan_down(_L12_BINS - 1, jnp.int32(_K))
        h1m = h1 - jnp.int32(1024)

        zero_hist(_HIST_WORDS)
        for ci in range(_NCHUNK):
            pltpu.sync_copy(cam_ref.at[row, ci], chunk)

            def b2(i, carry):
                x = chunk[pl.ds(i * 16, 16)]
                key = _monotone_key(x)
                sel = lax.shift_right_arithmetic(key, 21) == h1m
                bin2 = lax.shift_right_arithmetic(key, 10) & jnp.int32(0x7FF)
                plsc.addupdate_scatter(hist, [bin2 * 16 + lane], ones,
                                       mask=sel)
                return carry
            lax.fori_loop(0, _CH // 16, b2, 0)
        h2, _ = scan_down(_L12_BINS - 1, jnp.int32(_K) - above1)
        top22 = (h1m << 11) | h2

        zero_hist(_L3_BINS * 16)
        sumv = zeros_f
        cntv = zeros_i
        for ci in range(_NCHUNK):
            pltpu.sync_copy(cam_ref.at[row, ci], chunk)

            def b3(i, carry):
                sv, cv = carry
                x = chunk[pl.ds(i * 16, 16)]
                key = _monotone_key(x)
                hi22 = lax.shift_right_arithmetic(key, 10)
                inbin = hi22 == top22
                above = hi22 > top22
                bin3 = key & jnp.int32(0x3FF)
                plsc.addupdate_scatter(hist, [bin3 * 16 + lane], ones,
                                       mask=inbin)
                sv = sv + jnp.where(above, x, zeros_f)
                cv = cv + jnp.where(above, ones, zeros_i)
                return (sv, cv)
            sumv, cntv = lax.fori_loop(0, _CH // 16, b3, (sumv, cntv))
        c_above = jnp.sum(cntv)
        sum_above = jnp.sum(sumv)
        k3 = jnp.int32(_K) - c_above

        def cond3(c):
            _, cum, _, _, _ = c
            return cum < k3

        def body3(c):
            b, cum, wsum, _, _ = c
            cnt = jnp.sum(hist[pl.ds(b * 16, 16)])
            val = _decode_key((top22 << 10) | b)
            return (b - 1, cum + cnt,
                    wsum + cnt.astype(jnp.float32) * val, cnt, val)

        _, cum, wsum, lastc, lastv = lax.while_loop(
            cond3, body3,
            (jnp.int32(_L3_BINS - 1), jnp.int32(0), jnp.float32(0.0),
             jnp.int32(0), jnp.float32(0.0)))
        ties = (k3 - (cum - lastc)).astype(jnp.float32)
        wsum_full = wsum - lastc.astype(jnp.float32) * lastv
        total = sum_above + wsum_full + ties * lastv
        return total * jnp.float32(1.0 / _K)

    m0 = run_row(wid * _ROWS_PER_TILE)
    m1 = run_row(wid * _ROWS_PER_TILE + 1)
    outv[...] = jnp.where(lane == 0, m0, jnp.where(lane == 1, m1, zeros_f))
    pltpu.sync_copy(outv, out_ref.at[wid])


@functools.partial(
    pl.kernel,
    mesh=plsc.VectorSubcoreMesh(core_axis_name="c", subcore_axis_name="s"),
    out_type=jax.ShapeDtypeStruct((_NTILES, 16), jnp.float32),
    scratch_types=[
        pltpu.VMEM((_HIST_WORDS,), jnp.int32),
        pltpu.VMEM((_CH,), jnp.float32),
        pltpu.VMEM((16,), jnp.float32),
    ],
    compiler_params=pltpu.CompilerParams(needs_layout_passes=False),
)
def _sc_topk(cam_ref, out_ref, hist, chunk, outv):
    _sc_body(cam_ref, out_ref, hist, chunk, outv)


@jax.jit
def kernel(cam):
    batch, ncls, h, w = cam.shape
    rows = cam.reshape(_NROWS, _NCHUNK, _CH)
    out = _sc_topk(rows)
    return out[:, :_ROWS_PER_TILE].reshape(batch, ncls)


# speedup vs baseline: 1.3436x; 1.3436x over previous
import functools

import jax
import jax.numpy as jnp
from jax import lax
from jax.experimental import pallas as pl
from jax.experimental.pallas import tpu as pltpu
from jax.experimental.pallas import tpu_sc as plsc

_PERCENT_T = 0.02
_N = 512 * 512
_K = int(round(_N * _PERCENT_T))
_NROWS = 64
_NTILES = 32
_ROWS_PER_TILE = _NROWS // _NTILES
_CH = 32768
_NCHUNK = _N // _CH
_L12_BINS = 2048
_L3_BINS = 1024
_HIST_WORDS = _L12_BINS * 16
_UNROLL = 8


def _monotone_key(x):
    b = lax.bitcast_convert_type(x, jnp.int32)
    return jnp.where(b < 0, b ^ jnp.int32(0x7FFFFFFF), b)


def _decode_keys(keys):
    fb = jnp.where(keys < 0, keys ^ jnp.int32(0x7FFFFFFF), keys)
    return lax.bitcast_convert_type(fb, jnp.float32)


def _sc_body(cam_ref, out_ref, hist, chunks, outv, sem):
    cid = lax.axis_index("c")
    sid = lax.axis_index("s")
    wid = sid * 2 + cid
    lane = lax.iota(jnp.int32, 16)
    ones = jnp.ones((16,), jnp.int32)
    zeros_i = jnp.zeros((16,), jnp.int32)
    zeros_f = jnp.zeros((16,), jnp.float32)
    laneoff12 = lane * jnp.int32(_L12_BINS)
    laneoff3 = lane * jnp.int32(_L3_BINS)

    def zero_hist(nwords):
        def zb(i, carry):
            for j in range(_UNROLL):
                hist[pl.ds(i * (16 * _UNROLL) + j * 16, 16)] = zeros_i
            return carry
        lax.fori_loop(0, nwords // (16 * _UNROLL), zb, 0)

    def group_counts(g, nbins):
        acc = hist[pl.ds(g * 16, 16)]
        for l in range(1, 16):
            acc = acc + hist[pl.ds(l * nbins + g * 16, 16)]
        return acc

    def stream_pass(row, inner, carry):
        pend = pltpu.async_copy(cam_ref.at[row, 0], chunks[0], sem)
        for ci in range(_NCHUNK):
            pend.wait()
            if ci + 1 < _NCHUNK:
                pend = pltpu.async_copy(
                    cam_ref.at[row, ci + 1], chunks[(ci + 1) % 2], sem)
            carry = inner(chunks[ci % 2], carry)
        return carry

    def find_bin(nbins, target):
        def cond(c):
            _, cum, _ = c
            return cum < target

        def body(c):
            g, cum, _ = c
            cnt16 = group_counts(g, nbins)
            return (g - 1, cum + jnp.sum(cnt16), cnt16)

        g, cum, cnt16 = lax.while_loop(
            cond, body,
            (jnp.int32(nbins // 16 - 1), jnp.int32(0), zeros_i))
        gc = g + 1
        cum_bg = cum - jnp.sum(cnt16)
        srev = lax.rev(cnt16, (0,))
        csuf = plsc.cumsum(srev)
        below = jnp.where(cum_bg + csuf >= target, zeros_i, ones)
        j = jnp.sum(below)
        cj = jnp.sum(jnp.where(lane == j, csuf, zeros_i))
        sj = jnp.sum(jnp.where(lane == j, srev, zeros_i))
        h = gc * 16 + (jnp.int32(15) - j)
        above = cum_bg + cj - sj
        return h, above

    def run_row(row):
        zero_hist(_HIST_WORDS)

        def p1(buf, carry):
            def b1(ii, c):
                for j in range(_UNROLL):
                    x = buf[pl.ds(ii * (16 * _UNROLL) + j * 16, 16)]
                    key = _monotone_key(x)
                    bin1 = (lax.shift_right_arithmetic(key, 21)
                            + jnp.int32(1024))
                    plsc.addupdate_scatter(hist, [laneoff12 + bin1], ones)
                return c
            return lax.fori_loop(0, _CH // (16 * _UNROLL), b1, carry)

        stream_pass(row, p1, 0)
        h1, above1 = find_bin(_L12_BINS, jnp.int32(_K))
        h1m = h1 - jnp.int32(1024)

        zero_hist(_HIST_WORDS)

        def p2(buf, carry):
            def b2(ii, c):
                for j in range(_UNROLL):
                    x = buf[pl.ds(ii * (16 * _UNROLL) + j * 16, 16)]
                    key = _monotone_key(x)
                    sel = lax.shift_right_arithmetic(key, 21) == h1m
                    bin2 = (lax.shift_right_arithmetic(key, 10)
                            & jnp.int32(0x7FF))
                    plsc.addupdate_scatter(hist, [laneoff12 + bin2], ones,
                                           mask=sel)
                return c
            return lax.fori_loop(0, _CH // (16 * _UNROLL), b2, carry)

        stream_pass(row, p2, 0)
        h2, _ = find_bin(_L12_BINS, jnp.int32(_K) - above1)
        top22 = (h1m << 11) | h2

        zero_hist(_L3_BINS * 16)

        def p3(buf, carry):
            def b3(ii, c):
                sv, cv = c
                for j in range(_UNROLL):
                    x = buf[pl.ds(ii * (16 * _UNROLL) + j * 16, 16)]
                    key = _monotone_key(x)
                    hi22 = lax.shift_right_arithmetic(key, 10)
                    inbin = hi22 == top22
                    above = hi22 > top22
                    bin3 = key & jnp.int32(0x3FF)
                    plsc.addupdate_scatter(hist, [laneoff3 + bin3], ones,
                                           mask=inbin)
                    sv = sv + jnp.where(above, x, zeros_f)
                    cv = cv + jnp.where(above, ones, zeros_i)
                return (sv, cv)
            return lax.fori_loop(0, _CH // (16 * _UNROLL), b3, carry)

        sumv, cntv = stream_pass(row, p3, (zeros_f, zeros_i))
        c_above = jnp.sum(cntv)
        sum_above = jnp.sum(sumv)
        k3 = jnp.int32(_K) - c_above

        base22 = top22 << 10

        def vals_of_group(g):
            keys = base22 | (g * 16 + lane)
            return _decode_keys(keys)

        def cond3(c):
            _, cum, _, _ = c
            return cum < k3

        def body3(c):
            g, cum, wsumv, _ = c
            cnt16 = group_counts(g, _L3_BINS)
            wsumv = wsumv + cnt16.astype(jnp.float32) * vals_of_group(g)
            return (g - 1, cum + jnp.sum(cnt16), wsumv, cnt16)

        g, cum, wsumv, cnt16 = lax.while_loop(
            cond3, body3,
            (jnp.int32(_L3_BINS // 16 - 1), jnp.int32(0), zeros_f, zeros_i))
        gc = g + 1
        cum_bg = cum - jnp.sum(cnt16)
        vals16 = vals_of_group(gc)
        w_groups_above = jnp.sum(wsumv) - jnp.sum(
            cnt16.astype(jnp.float32) * vals16)
        srev = lax.rev(cnt16, (0,))
        vrev = lax.rev(vals16, (0,))
        csuf = plsc.cumsum(srev)
        below = jnp.where(cum_bg + csuf >= k3, zeros_i, ones)
        j = jnp.sum(below)
        cj = jnp.sum(jnp.where(lane == j, csuf, zeros_i))
        sj = jnp.sum(jnp.where(lane == j, srev, zeros_i))
        val_l = jnp.sum(jnp.where(lane == j, vrev, zeros_f))
        wpart = jnp.sum(jnp.where(lane < j,
                                  srev.astype(jnp.float32) * vrev, zeros_f))
        ties = (k3 - (cum_bg + cj - sj)).astype(jnp.float32)
        total = sum_above + w_groups_above + wpart + ties * val_l
        return total * jnp.float32(1.0 / _K)

    m0 = run_row(wid * _ROWS_PER_TILE)
    m1 = run_row(wid * _ROWS_PER_TILE + 1)
    outv[...] = jnp.where(lane == 0, m0, jnp.where(lane == 1, m1, zeros_f))
    pltpu.sync_copy(outv, out_ref.at[wid])


@functools.partial(
    pl.kernel,
    mesh=plsc.VectorSubcoreMesh(core_axis_name="c", subcore_axis_name="s"),
    out_type=jax.ShapeDtypeStruct((_NTILES, 16), jnp.float32),
    scratch_types=[
        pltpu.VMEM((_HIST_WORDS,), jnp.int32),
        pltpu.VMEM((_CH,), jnp.float32),
        pltpu.VMEM((_CH,), jnp.float32),
        pltpu.VMEM((16,), jnp.float32),
        pltpu.SemaphoreType.DMA,
    ],
    compiler_params=pltpu.CompilerParams(needs_layout_passes=False),
)
def _sc_topk(cam_ref, out_ref, hist, chunk_a, chunk_b, outv, sem):
    _sc_body(cam_ref, out_ref, hist, (chunk_a, chunk_b), outv, sem)


@jax.jit
def kernel(cam):
    batch, ncls, h, w = cam.shape
    rows = cam.reshape(_NROWS, _NCHUNK, _CH)
    out = _sc_topk(rows)
    return out[:, :_ROWS_PER_TILE].reshape(batch, ncls)
